# baseline (speedup 1.0000x reference)
import functools

import jax
import jax.numpy as jnp
from jax import lax
from jax.experimental import pallas as pl
from jax.experimental.pallas import tpu as pltpu
from jax.experimental.pallas import tpu_sc as plsc

B = 16384
EMB = 32
TBLK = 128
K = 8
GRP = 16
BLK = 2048


def _sc_info():
    info = plsc.get_sparse_core_info()
    return info.num_cores, info.num_subcores


@functools.lru_cache(maxsize=1)
def _make_sc_gather():
    nc, ns = _sc_info()
    nw = nc * ns
    bpw = B // nw
    nblk = bpw // TBLK
    ngrp = bpw // GRP
    np_bound = bpw // K + 2

    mesh = plsc.VectorSubcoreMesh(core_axis_name="c", subcore_axis_name="s")

    @functools.partial(
        pl.kernel,
        mesh=mesh,
        out_type=jax.ShapeDtypeStruct((EMB, B), jnp.float32),
        scratch_types=[
            pltpu.VMEM((bpw,), jnp.int32),
            pltpu.VMEM((B,), jnp.int32),
            pltpu.VMEM((2, K, EMB, TBLK), jnp.float32),
            pltpu.VMEM((nblk, EMB, TBLK), jnp.float32),
            pltpu.SMEM((bpw,), jnp.int32),
            pltpu.SMEM((bpw,), jnp.int32),
            pltpu.SMEM((np_bound,), jnp.int32),
            [pltpu.SemaphoreType.DMA] * K,
            [pltpu.SemaphoreType.DMA] * K,
        ],
        compiler_params=pltpu.CompilerParams(
            use_tc_tiling_on_sc=True, needs_layout_passes=False),
    )
    def sc_gather(pk_hbm, id_hbm, tab_hbm, out_hbm,
                  idx_v, idfull_v, fbuf, acc, uniq_s, pk_s, p_s,
                  sems0, sems1):
        wid = lax.axis_index("s") * nc + lax.axis_index("c")
        base = wid * bpw
        pltpu.sync_copy(pk_hbm.at[pl.ds(base, bpw)], idx_v)
        pltpu.sync_copy(id_hbm, idfull_v)

        rowv = lax.iota(jnp.int32, 16)
        rowv16 = rowv + 16

        def scan_ids(idx_v):
            for q in range(np_bound):
                p_s[q] = bpw

            def sg_body(sg, carry):
                prev, cnt = carry
                vec = idx_v[pl.ds(sg * GRP, GRP)]
                rtv = vec // 16384
                posv = vec - rtv * 16384
                idv = plsc.load_gather(idfull_v, [posv])
                cvv = idv - (idv // TBLK) * TBLK
                for j in range(GRP):
                    rt = rtv[j]
                    c = cvv[j]
                    fresh = rt != prev
                    cnt = cnt + jnp.where(fresh, 1, 0)
                    um = cnt - 1
                    uniq_s[um] = rt
                    pk_s[sg * GRP + j] = um * TBLK + c

                    @pl.when(jnp.logical_and(fresh, lax.rem(um, K) == 0))
                    def _():
                        p_s[um // K] = sg * GRP + j

                    prev = rt
                return (prev, cnt)

            _, cnt = lax.fori_loop(0, ngrp, sg_body,
                                   (jnp.int32(-1), jnp.int32(0)))
            return cnt

        def issue_grp(tab, cnt, g, half):
            sems = sems0 if half == 0 else sems1
            for k in range(K):
                ug = jnp.minimum(g * K + k, cnt - 1)
                rt = uniq_s[ug]
                c0 = pl.multiple_of(rt * TBLK, TBLK)
                pltpu.async_copy(tab.at[:, pl.ds(c0, TBLK)],
                                 fbuf.at[half, k], sems[k])

        def wait_grp(tab, half):
            sems = sems0 if half == 0 else sems1
            for k in range(K):
                pltpu.make_async_copy(tab.at[:, pl.ds(0, TBLK)],
                                      fbuf.at[half, k], sems[k]).wait()

        def gather_table(tab, idx_v, out_hbm):
            cnt = scan_ids(idx_v)
            cnt = jnp.maximum(cnt, 1)
            issue_grp(tab, cnt, 0, 0)
            issue_grp(tab, cnt, 1, 1)
            ngroups = (cnt + (K - 1)) // K
            ng2 = (ngroups + 1) // 2

            def body(g2, _):
                for half in (0, 1):
                    ga = 2 * g2 + half
                    wait_grp(tab, half)
                    lo = p_s[ga]
                    hi = p_s[ga + 1]

                    def ext(jj, _):
                        pk = pk_s[jj]
                        um = pk // TBLK
                        c = pk - um * TBLK
                        slot = um - ga * K
                        slv = jnp.full((16,), slot, jnp.int32)
                        cv = jnp.full((16,), c, jnp.int32)
                        bv = jnp.full((16,), jj // TBLK, jnp.int32)
                        colv = jnp.full((16,), lax.rem(jj, TBLK), jnp.int32)
                        hb = fbuf.at[half]
                        v0 = plsc.load_gather(hb, [slv, rowv, cv])
                        v1 = plsc.load_gather(hb, [slv, rowv16, cv])
                        plsc.store_scatter(acc, [bv, rowv, colv], v0)
                        plsc.store_scatter(acc, [bv, rowv16, colv], v1)
                        return ()

                    lax.fori_loop(lo, hi, ext, ())
                    issue_grp(tab, cnt, 2 * g2 + 2 + half, half)
                return ()

            lax.fori_loop(0, ng2, body, ())
            wait_grp(tab, 0)
            wait_grp(tab, 1)
            for b in range(nblk):
                pltpu.sync_copy(acc.at[b],
                                out_hbm.at[:, pl.ds(base + b * TBLK, TBLK)])

        gather_table(tab_hbm, idx_v, out_hbm)

    return sc_gather


@functools.lru_cache(maxsize=1)
def _make_sc_unperm():
    nc, ns = _sc_info()
    nw = nc * ns
    bpw = B // nw
    nq = bpw // TBLK
    nsg = bpw // GRP

    mesh = plsc.VectorSubcoreMesh(core_axis_name="c", subcore_axis_name="s")

    @functools.partial(
        pl.kernel,
        mesh=mesh,
        out_type=[
            jax.ShapeDtypeStruct((B, EMB), jnp.float32),
            jax.ShapeDtypeStruct((B, EMB), jnp.float32),
        ],
        scratch_types=[
            pltpu.VMEM((EMB, bpw), jnp.float32),
            pltpu.VMEM((EMB, bpw), jnp.float32),
            pltpu.VMEM((bpw, EMB), jnp.float32),
            pltpu.VMEM((bpw, EMB), jnp.float32),
            pltpu.VMEM((nq, TBLK), jnp.int32),
            pltpu.VMEM((nq, TBLK), jnp.int32),
            pltpu.SemaphoreType.DMA,
        ],
        compiler_params=pltpu.CompilerParams(
            use_tc_tiling_on_sc=False, needs_layout_passes=False),
    )
    def sc_unperm(us_hbm, up_hbm, is_hbm, ip_hbm, u_out, i_out,
                  ucols, icols, urows, irows, upb, ipb, sem):
        wid = lax.axis_index("s") * nc + lax.axis_index("c")
        base = wid * bpw
        cps = [
            pltpu.async_copy(us_hbm.at[:, pl.ds(base, bpw)], ucols, sem),
            pltpu.async_copy(is_hbm.at[:, pl.ds(base, bpw)], icols, sem),
        ]
        for q in range(nq):
            cps.append(pltpu.async_copy(
                up_hbm.at[pl.ds(base + q * TBLK, TBLK)], upb.at[q], sem))
            cps.append(pltpu.async_copy(
                ip_hbm.at[pl.ds(base + q * TBLK, TBLK)], ipb.at[q], sem))
        for cp in cps:
            cp.wait()
        for q in range(nq):
            for l in range(TBLK // GRP):
                pv = upb[q, pl.ds(l * GRP, GRP)]
                upb[q, pl.ds(l * GRP, GRP)] = pv - (pv // 16384) * 16384
                pv = ipb[q, pl.ds(l * GRP, GRP)]
                ipb[q, pl.ds(l * GRP, GRP)] = pv - (pv // 16384) * 16384

        rowv = lax.iota(jnp.int32, 16)

        def tgroup(sg, _):
            s0 = sg * GRP
            for f in range(EMB):
                uv = ucols[f, pl.ds(s0, GRP)]
                iv = icols[f, pl.ds(s0, GRP)]
                plsc.store_scatter(urows, [rowv + s0, jnp.full((16,), f, jnp.int32)], uv)
                plsc.store_scatter(irows, [rowv + s0, jnp.full((16,), f, jnp.int32)], iv)
            return ()

        lax.fori_loop(0, nsg, tgroup, (), unroll=False)

        outs = []
        for q in range(nq):
            outs.append(pltpu.async_copy(
                urows.at[pl.ds(q * TBLK, TBLK), :], u_out.at[upb.at[q]], sem))
            outs.append(pltpu.async_copy(
                irows.at[pl.ds(q * TBLK, TBLK), :], i_out.at[ipb.at[q]], sem))
        for cp in outs:
            cp.wait()

    return sc_unperm


def _mlp_body(u_ref, i_ref, w1a_ref, w1b_ref, b1_ref, w2_ref, b2_ref,
              w3_ref, b3_ref, out_ref):
    h = jnp.dot(u_ref[...], w1a_ref[...], preferred_element_type=jnp.float32)
    h = h + jnp.dot(i_ref[...], w1b_ref[...], preferred_element_type=jnp.float32)
    h = jnp.maximum(h + b1_ref[...], 0.0)
    h = jnp.dot(h, w2_ref[...], preferred_element_type=jnp.float32)
    h = jnp.maximum(h + b2_ref[...], 0.0)
    out_ref[...] = jnp.sum(h * w3_ref[...], axis=1) + b3_ref[0, 0]


def _mlp(u, i, W1, b1, W2, b2, W3, b3):
    grid = (B // BLK,)
    bcast = lambda s: pl.BlockSpec(s, lambda b: (0, 0))
    return pl.pallas_call(
        _mlp_body,
        grid=grid,
        in_specs=[
            pl.BlockSpec((BLK, EMB), lambda b: (b, 0)),
            pl.BlockSpec((BLK, EMB), lambda b: (b, 0)),
            bcast((EMB, 32)),
            bcast((EMB, 32)),
            bcast((1, 32)),
            bcast((32, 16)),
            bcast((1, 16)),
            bcast((1, 16)),
            bcast((1, 1)),
        ],
        out_specs=pl.BlockSpec((BLK,), lambda b: (b,)),
        out_shape=jax.ShapeDtypeStruct((B,), jnp.float32),
    )(u, i, W1[:EMB], W1[EMB:], b1.reshape(1, 32), W2, b2.reshape(1, 16),
      W3.reshape(1, 16), b3.reshape(1, 1))


def kernel(user_id, item_id, user_table, item_table, W1, b1, W2, b2, W3, b3):
    uid = user_id.astype(jnp.int32)
    iid = item_id.astype(jnp.int32)
    pos = lax.iota(jnp.int32, B)
    upk = lax.sort((uid // TBLK) * 16384 + pos)
    ipk = lax.sort((iid // TBLK) * 16384 + pos)
    sc_gather = _make_sc_gather()
    us_cols = sc_gather(upk, uid, user_table.T)
    is_cols = sc_gather(ipk, iid, item_table.T)
    u, i = _make_sc_unperm()(us_cols, upk, is_cols, ipk)
    return _mlp(u, i, W1, b1, W2, b2, W3, b3)


# speedup vs baseline: 1.0435x; 1.0435x over previous
import functools

import jax
import jax.numpy as jnp
from jax import lax
from jax.experimental import pallas as pl
from jax.experimental.pallas import tpu as pltpu
from jax.experimental.pallas import tpu_sc as plsc

B = 16384
EMB = 32
TBLK = 128
K = 8
GRP = 16
BLK = 2048


def _sc_info():
    info = plsc.get_sparse_core_info()
    return info.num_cores, info.num_subcores


@functools.lru_cache(maxsize=1)
def _make_sc_gather():
    nc, ns = _sc_info()
    nw = nc * ns
    bpw = B // nw
    nblk = bpw // TBLK
    ngrp = bpw // GRP
    np_bound = bpw // K + 2

    mesh = plsc.VectorSubcoreMesh(core_axis_name="c", subcore_axis_name="s")

    @functools.partial(
        pl.kernel,
        mesh=mesh,
        out_type=jax.ShapeDtypeStruct((EMB, B), jnp.float32),
        scratch_types=[
            pltpu.VMEM((bpw,), jnp.int32),
            pltpu.VMEM((2, K, EMB, TBLK), jnp.float32),
            pltpu.VMEM((nblk, EMB, TBLK), jnp.float32),
            pltpu.SMEM((bpw,), jnp.int32),
            pltpu.SMEM((bpw,), jnp.int32),
            pltpu.SMEM((np_bound,), jnp.int32),
            [pltpu.SemaphoreType.DMA] * K,
            [pltpu.SemaphoreType.DMA] * K,
        ],
        compiler_params=pltpu.CompilerParams(
            use_tc_tiling_on_sc=True, needs_layout_passes=False),
    )
    def sc_gather(id_hbm, tab_hbm, out_hbm,
                  idx_v, fbuf, acc, uniq_s, pk_s, p_s, sems0, sems1):
        wid = lax.axis_index("s") * nc + lax.axis_index("c")
        base = wid * bpw
        pltpu.sync_copy(id_hbm.at[pl.ds(base, bpw)], idx_v)

        rowv = lax.iota(jnp.int32, 16)
        rowv16 = rowv + 16

        def scan_ids(idx_v):
            for q in range(np_bound):
                p_s[q] = bpw

            def sg_body(sg, carry):
                prev, cnt = carry
                vec = idx_v[pl.ds(sg * GRP, GRP)]
                for j in range(GRP):
                    r = vec[j]
                    rt = r // TBLK
                    c = r - rt * TBLK
                    fresh = rt != prev
                    cnt = cnt + jnp.where(fresh, 1, 0)
                    um = cnt - 1
                    uniq_s[um] = rt
                    pk_s[sg * GRP + j] = um * TBLK + c

                    @pl.when(jnp.logical_and(fresh, lax.rem(um, K) == 0))
                    def _():
                        p_s[um // K] = sg * GRP + j

                    prev = rt
                return (prev, cnt)

            _, cnt = lax.fori_loop(0, ngrp, sg_body,
                                   (jnp.int32(-1), jnp.int32(0)))
            return cnt

        def issue_grp(tab, cnt, g, half):
            sems = sems0 if half == 0 else sems1
            for k in range(K):
                ug = jnp.minimum(g * K + k, cnt - 1)
                rt = uniq_s[ug]
                c0 = pl.multiple_of(rt * TBLK, TBLK)
                pltpu.async_copy(tab.at[:, pl.ds(c0, TBLK)],
                                 fbuf.at[half, k], sems[k])

        def wait_grp(tab, half):
            sems = sems0 if half == 0 else sems1
            for k in range(K):
                pltpu.make_async_copy(tab.at[:, pl.ds(0, TBLK)],
                                      fbuf.at[half, k], sems[k]).wait()

        def gather_table(tab, idx_v, out_hbm):
            cnt = scan_ids(idx_v)
            cnt = jnp.maximum(cnt, 1)
            issue_grp(tab, cnt, 0, 0)
            issue_grp(tab, cnt, 1, 1)
            ngroups = (cnt + (K - 1)) // K
            ng2 = (ngroups + 1) // 2

            def body(g2, _):
                for half in (0, 1):
                    ga = 2 * g2 + half
                    wait_grp(tab, half)
                    lo = p_s[ga]
                    hi = p_s[ga + 1]

                    def ext(jj, _):
                        pk = pk_s[jj]
                        um = pk // TBLK
                        c = pk - um * TBLK
                        slot = um - ga * K
                        slv = jnp.full((16,), slot, jnp.int32)
                        cv = jnp.full((16,), c, jnp.int32)
                        bv = jnp.full((16,), jj // TBLK, jnp.int32)
                        colv = jnp.full((16,), lax.rem(jj, TBLK), jnp.int32)
                        hb = fbuf.at[half]
                        v0 = plsc.load_gather(hb, [slv, rowv, cv])
                        v1 = plsc.load_gather(hb, [slv, rowv16, cv])
                        plsc.store_scatter(acc, [bv, rowv, colv], v0)
                        plsc.store_scatter(acc, [bv, rowv16, colv], v1)
                        return ()

                    lax.fori_loop(lo, hi, ext, ())
                    issue_grp(tab, cnt, 2 * g2 + 2 + half, half)
                return ()

            lax.fori_loop(0, ng2, body, ())
            wait_grp(tab, 0)
            wait_grp(tab, 1)
            for b in range(nblk):
                pltpu.sync_copy(acc.at[b],
                                out_hbm.at[:, pl.ds(base + b * TBLK, TBLK)])

        gather_table(tab_hbm, idx_v, out_hbm)

    return sc_gather


@functools.lru_cache(maxsize=1)
def _make_sc_unperm():
    nc, ns = _sc_info()
    nw = nc * ns
    bpw = B // nw
    nq = bpw // TBLK
    nsg = bpw // GRP

    mesh = plsc.VectorSubcoreMesh(core_axis_name="c", subcore_axis_name="s")

    @functools.partial(
        pl.kernel,
        mesh=mesh,
        out_type=[
            jax.ShapeDtypeStruct((B, EMB), jnp.float32),
            jax.ShapeDtypeStruct((B, EMB), jnp.float32),
        ],
        scratch_types=[
            pltpu.VMEM((EMB, bpw), jnp.float32),
            pltpu.VMEM((EMB, bpw), jnp.float32),
            pltpu.VMEM((bpw, EMB), jnp.float32),
            pltpu.VMEM((bpw, EMB), jnp.float32),
            pltpu.VMEM((nq, TBLK), jnp.int32),
            pltpu.VMEM((nq, TBLK), jnp.int32),
            pltpu.SemaphoreType.DMA,
        ],
        compiler_params=pltpu.CompilerParams(
            use_tc_tiling_on_sc=False, needs_layout_passes=False),
    )
    def sc_unperm(us_hbm, up_hbm, is_hbm, ip_hbm, u_out, i_out,
                  ucols, icols, urows, irows, upb, ipb, sem):
        wid = lax.axis_index("s") * nc + lax.axis_index("c")
        base = wid * bpw
        cps = [
            pltpu.async_copy(us_hbm.at[:, pl.ds(base, bpw)], ucols, sem),
            pltpu.async_copy(is_hbm.at[:, pl.ds(base, bpw)], icols, sem),
        ]
        for q in range(nq):
            cps.append(pltpu.async_copy(
                up_hbm.at[pl.ds(base + q * TBLK, TBLK)], upb.at[q], sem))
            cps.append(pltpu.async_copy(
                ip_hbm.at[pl.ds(base + q * TBLK, TBLK)], ipb.at[q], sem))
        for cp in cps:
            cp.wait()

        rowv = lax.iota(jnp.int32, 16)

        def tgroup(sg, _):
            s0 = sg * GRP
            for f in range(EMB):
                uv = ucols[f, pl.ds(s0, GRP)]
                iv = icols[f, pl.ds(s0, GRP)]
                plsc.store_scatter(urows, [rowv + s0, jnp.full((16,), f, jnp.int32)], uv)
                plsc.store_scatter(irows, [rowv + s0, jnp.full((16,), f, jnp.int32)], iv)
            return ()

        lax.fori_loop(0, nsg, tgroup, (), unroll=False)

        outs = []
        for q in range(nq):
            outs.append(pltpu.async_copy(
                urows.at[pl.ds(q * TBLK, TBLK), :], u_out.at[upb.at[q]], sem))
            outs.append(pltpu.async_copy(
                irows.at[pl.ds(q * TBLK, TBLK), :], i_out.at[ipb.at[q]], sem))
        for cp in outs:
            cp.wait()

    return sc_unperm


def _mlp_body(u_ref, i_ref, w1a_ref, w1b_ref, b1_ref, w2_ref, b2_ref,
              w3_ref, b3_ref, out_ref):
    h = jnp.dot(u_ref[...], w1a_ref[...], preferred_element_type=jnp.float32)
    h = h + jnp.dot(i_ref[...], w1b_ref[...], preferred_element_type=jnp.float32)
    h = jnp.maximum(h + b1_ref[...], 0.0)
    h = jnp.dot(h, w2_ref[...], preferred_element_type=jnp.float32)
    h = jnp.maximum(h + b2_ref[...], 0.0)
    out_ref[...] = jnp.sum(h * w3_ref[...], axis=1) + b3_ref[0, 0]


def _mlp(u, i, W1, b1, W2, b2, W3, b3):
    grid = (B // BLK,)
    bcast = lambda s: pl.BlockSpec(s, lambda b: (0, 0))
    return pl.pallas_call(
        _mlp_body,
        grid=grid,
        in_specs=[
            pl.BlockSpec((BLK, EMB), lambda b: (b, 0)),
            pl.BlockSpec((BLK, EMB), lambda b: (b, 0)),
            bcast((EMB, 32)),
            bcast((EMB, 32)),
            bcast((1, 32)),
            bcast((32, 16)),
            bcast((1, 16)),
            bcast((1, 16)),
            bcast((1, 1)),
        ],
        out_specs=pl.BlockSpec((BLK,), lambda b: (b,)),
        out_shape=jax.ShapeDtypeStruct((B,), jnp.float32),
    )(u, i, W1[:EMB], W1[EMB:], b1.reshape(1, 32), W2, b2.reshape(1, 16),
      W3.reshape(1, 16), b3.reshape(1, 1))


def kernel(user_id, item_id, user_table, item_table, W1, b1, W2, b2, W3, b3):
    uid = user_id.astype(jnp.int32)
    iid = item_id.astype(jnp.int32)
    pos = lax.iota(jnp.int32, B)
    uid_s, uperm = lax.sort_key_val(uid, pos)
    iid_s, iperm = lax.sort_key_val(iid, pos)
    sc_gather = _make_sc_gather()
    us_cols = sc_gather(uid_s, user_table.T)
    is_cols = sc_gather(iid_s, item_table.T)
    u, i = _make_sc_unperm()(us_cols, uperm, is_cols, iperm)
    return _mlp(u, i, W1, b1, W2, b2, W3, b3)
